# Initial kernel scaffold; baseline (speedup 1.0000x reference)
import jax
import jax.numpy as jnp
from jax.experimental import pallas as pl


def kernel(flat, cu_seqlens):
    raise NotImplementedError("write your pallas kernel here")


# speedup vs baseline: 1.9790x; 1.9790x over previous
import functools

import jax
import jax.numpy as jnp
from jax import lax
from jax.experimental import pallas as pl
from jax.experimental.pallas import tpu as pltpu
from jax.experimental.pallas import tpu_sc as plsc

_B = 16
_MAX_LEN = 2048
_TOTAL = 16384
_D = 256
_NW = 32
_R = _B * _MAX_LEN // _NW
_CH = 128
_NCH = _R // _CH
_TAIL = (64, 32, 16, 8, 4, 2, 1)

_mesh = plsc.VectorSubcoreMesh(core_axis_name="c", subcore_axis_name="s")


@functools.partial(
    pl.kernel,
    mesh=_mesh,
    compiler_params=pltpu.CompilerParams(use_tc_tiling_on_sc=False,
                                          needs_layout_passes=False),
    out_type=jax.ShapeDtypeStruct((_B * _MAX_LEN, _D), jnp.float32),
    scratch_types=[
        pltpu.VMEM((_CH, _D), jnp.float32),
        pltpu.VMEM((_CH, _D), jnp.float32),
        pltpu.VMEM((_CH, _D), jnp.float32),
        pltpu.VMEM((16,), jnp.int32),
        pltpu.VMEM((16,), jnp.int32),
    ],
)
def _pad_kernel(flat_hbm, cu_hbm, len_hbm, zeros_hbm, out_hbm,
                buf0, buf1, zbuf, cu_v, len_v):
    c = lax.axis_index("c")
    s = lax.axis_index("s")
    wid = s * 2 + c
    b = wid // 2
    p0 = (wid % 2) * _R

    pltpu.sync_copy(cu_hbm, cu_v)
    pltpu.sync_copy(len_hbm, len_v)
    pltpu.sync_copy(zeros_hbm, zbuf)

    lanes = lax.iota(jnp.int32, 16)
    sel = lanes == b
    cu_b = jnp.sum(jnp.where(sel, cu_v[...], 0))
    len_b = jnp.sum(jnp.where(sel, len_v[...], 0))

    nd = jnp.clip(len_b - p0, 0, _R)
    npad = _R - nd
    base = wid * _R
    src0 = cu_b + p0

    nfull = nd // _CH
    for j in range(_NCH):
        @pl.when(j < nfull)
        def _(j=j):
            buf = buf0 if j % 2 == 0 else buf1
            pltpu.sync_copy(flat_hbm.at[pl.ds(src0 + j * _CH, _CH)], buf)
            pltpu.sync_copy(buf, out_hbm.at[pl.ds(base + j * _CH, _CH)])

    rem = nd % _CH
    off = nfull * _CH
    for sz in _TAIL:
        bit = (rem // sz) % 2
        @pl.when(bit == 1)
        def _(sz=sz, off=off):
            pltpu.sync_copy(flat_hbm.at[pl.ds(src0 + off, sz)],
                            buf0.at[pl.ds(0, sz)])
            pltpu.sync_copy(buf0.at[pl.ds(0, sz)],
                            out_hbm.at[pl.ds(base + off, sz)])
        off = off + bit * sz

    nzfull = npad // _CH
    z0 = base + nd
    for j in range(_NCH):
        @pl.when(j < nzfull)
        def _(j=j):
            pltpu.sync_copy(zbuf, out_hbm.at[pl.ds(z0 + j * _CH, _CH)])

    zrem = npad % _CH
    zoff = z0 + nzfull * _CH
    for sz in _TAIL:
        bit = (zrem // sz) % 2
        @pl.when(bit == 1)
        def _(sz=sz, zoff=zoff):
            pltpu.sync_copy(zbuf.at[pl.ds(0, sz)],
                            out_hbm.at[pl.ds(zoff, sz)])
        zoff = zoff + bit * sz


def kernel(flat, cu_seqlens):
    cu16 = cu_seqlens[:_B].astype(jnp.int32)
    lengths_i32 = (cu_seqlens[1:] - cu_seqlens[:-1]).astype(jnp.int32)
    zeros = jnp.zeros((_CH, _D), jnp.float32)
    out = _pad_kernel(flat, cu16, lengths_i32, zeros)
    padded = out.reshape(_B, _MAX_LEN, _D)
    lengths = lengths_i32.astype(jnp.int64)
    return padded, lengths


# speedup vs baseline: 2.1965x; 1.1099x over previous
import functools

import jax
import jax.numpy as jnp
from jax import lax
from jax.experimental import pallas as pl
from jax.experimental.pallas import tpu as pltpu
from jax.experimental.pallas import tpu_sc as plsc

_B = 16
_MAX_LEN = 2048
_TOTAL = 16384
_D = 256
_NW = 32
_R = _B * _MAX_LEN // _NW
_CH = 64
_NCH = _R // _CH
_NB = 4
_TAIL = (32, 16, 8, 4, 2, 1)
_SLOT = {32: 0, 16: 32, 8: 48, 4: 56, 2: 60, 1: 62}

_mesh = plsc.VectorSubcoreMesh(core_axis_name="c", subcore_axis_name="s")


@functools.partial(
    pl.kernel,
    mesh=_mesh,
    compiler_params=pltpu.CompilerParams(use_tc_tiling_on_sc=False,
                                         needs_layout_passes=False),
    out_type=jax.ShapeDtypeStruct((_B * _MAX_LEN, _D), jnp.float32),
    scratch_types=(
        [pltpu.VMEM((_CH, _D), jnp.float32)] * _NB
        + [pltpu.VMEM((_CH, _D), jnp.float32)]
        + [pltpu.VMEM((_CH, _D), jnp.float32)]
        + [pltpu.VMEM((16,), jnp.int32)] * 2
        + [pltpu.SemaphoreType.DMA] * (3 + 2 * _NB + 3)
    ),
)
def _pad_kernel(flat_hbm, cu_hbm, len_hbm, zeros_hbm, out_hbm,
                rb0, rb1, rb2, rb3, tbuf, zbuf, cu_v, len_v,
                sem_cu, sem_len, sem_zb,
                sl0, sl1, sl2, sl3, ss0, ss1, ss2, ss3,
                sem_z, sem_tl, sem_ts):
    bufs = (rb0, rb1, rb2, rb3)
    sls = (sl0, sl1, sl2, sl3)
    sss = (ss0, ss1, ss2, ss3)

    c = lax.axis_index("c")
    s = lax.axis_index("s")
    wid = s * 2 + c
    b = wid // 2
    p0 = (wid % 2) * _R

    pltpu.async_copy(cu_hbm, cu_v, sem_cu)
    pltpu.async_copy(len_hbm, len_v, sem_len)
    pltpu.async_copy(zeros_hbm, zbuf, sem_zb)
    pltpu.make_async_copy(cu_hbm, cu_v, sem_cu).wait()
    pltpu.make_async_copy(len_hbm, len_v, sem_len).wait()

    lanes = lax.iota(jnp.int32, 16)
    sel = lanes == b
    cu_b = jnp.sum(jnp.where(sel, cu_v[...], 0))
    len_b = jnp.sum(jnp.where(sel, len_v[...], 0))

    nd = jnp.clip(len_b - p0, 0, _R)
    npad = _R - nd
    base = wid * _R
    src0 = cu_b + p0
    nfull = nd // _CH
    rem = nd % _CH
    nzfull = npad // _CH
    zrem = npad % _CH
    z0 = base + nd
    tsrc = src0 + nfull * _CH
    tdst = base + nfull * _CH

    def ld(j):
        return pltpu.make_async_copy(
            flat_hbm.at[pl.ds(src0 + j * _CH, _CH)], bufs[j % _NB],
            sls[j % _NB])

    def st(j):
        return pltpu.make_async_copy(
            bufs[j % _NB], out_hbm.at[pl.ds(base + j * _CH, _CH)],
            sss[j % _NB])

    for j in range(_NB):
        @pl.when(j < nfull)
        def _(j=j):
            ld(j).start()

    pltpu.make_async_copy(zeros_hbm, zbuf, sem_zb).wait()
    zdescs = []
    for j in range(_NCH):
        d = pltpu.make_async_copy(
            zbuf, out_hbm.at[pl.ds(z0 + j * _CH, _CH)], sem_z)
        zdescs.append((j < nzfull, d))
    zoff = z0 + nzfull * _CH
    for sz in _TAIL:
        bit = (zrem // sz) % 2
        d = pltpu.make_async_copy(
            zbuf.at[pl.ds(0, sz)], out_hbm.at[pl.ds(zoff, sz)], sem_z)
        zdescs.append((bit == 1, d))
        zoff = zoff + bit * sz
    for cond, d in zdescs:
        @pl.when(cond)
        def _(d=d):
            d.start()

    for j in range(_NCH):
        @pl.when(j < nfull)
        def _(j=j):
            ld(j).wait()
            st(j).start()
        if j + _NB < _NCH:
            @pl.when(j + _NB < nfull)
            def _(j=j):
                st(j).wait()
                ld(j + _NB).start()

    tdescs = []
    cum = 0
    for sz in _TAIL:
        bit = (rem // sz) % 2
        dl = pltpu.make_async_copy(
            flat_hbm.at[pl.ds(tsrc + cum, sz)],
            tbuf.at[pl.ds(_SLOT[sz], sz)], sem_tl)
        ds_ = pltpu.make_async_copy(
            tbuf.at[pl.ds(_SLOT[sz], sz)],
            out_hbm.at[pl.ds(tdst + cum, sz)], sem_ts)
        tdescs.append((bit == 1, dl, ds_))
        cum = cum + bit * sz
    for cond, dl, _ds in tdescs:
        @pl.when(cond)
        def _(dl=dl):
            dl.start()
    for cond, dl, _ds in tdescs:
        @pl.when(cond)
        def _(dl=dl):
            dl.wait()
    for cond, _dl, ds_ in tdescs:
        @pl.when(cond)
        def _(ds_=ds_):
            ds_.start()

    for j in range(_NCH):
        @pl.when(jnp.logical_and(j < nfull, j + _NB >= nfull))
        def _(j=j):
            st(j).wait()
    for cond, d in zdescs:
        @pl.when(cond)
        def _(d=d):
            d.wait()
    for cond, _dl, ds_ in tdescs:
        @pl.when(cond)
        def _(ds_=ds_):
            ds_.wait()


def kernel(flat, cu_seqlens):
    cu16 = cu_seqlens[:_B].astype(jnp.int32)
    lengths_i32 = (cu_seqlens[1:] - cu_seqlens[:-1]).astype(jnp.int32)
    zeros = jnp.zeros((_CH, _D), jnp.float32)
    out = _pad_kernel(flat, cu16, lengths_i32, zeros)
    padded = out.reshape(_B, _MAX_LEN, _D)
    lengths = lengths_i32.astype(jnp.int64)
    return padded, lengths


# speedup vs baseline: 4.4205x; 2.0125x over previous
import functools

import jax
import jax.numpy as jnp
from jax import lax
from jax.experimental import pallas as pl
from jax.experimental.pallas import tpu as pltpu
from jax.experimental.pallas import tpu_sc as plsc

_B = 16
_MAX_LEN = 2048
_TOTAL = 16384
_D = 256
_NW = 32
_R = _B * _MAX_LEN // _NW
_CH = 64
_NCH = _R // _CH
_NB = 4
_ZTAIL = (32, 16, 8)
_MTAIL = (64, 32, 16, 8)

_mesh = plsc.VectorSubcoreMesh(core_axis_name="c", subcore_axis_name="s")


def _m8(x):
    return pl.multiple_of(x, 8)


@functools.partial(
    pl.kernel,
    mesh=_mesh,
    compiler_params=pltpu.CompilerParams(needs_layout_passes=False),
    out_type=jax.ShapeDtypeStruct((_B * _MAX_LEN, _D), jnp.float32),
    scratch_types=(
        [pltpu.VMEM((_CH, _D), jnp.float32)] * _NB
        + [pltpu.VMEM((_CH, _D), jnp.float32)]
        + [pltpu.VMEM((_CH, _D), jnp.float32)]
        + [pltpu.VMEM((_CH,), jnp.int32)] * _NB
        + [pltpu.VMEM((_CH,), jnp.int32)]
        + [pltpu.VMEM((16,), jnp.int32)]
        + [pltpu.VMEM((16,), jnp.int32)] * 2
        + [pltpu.SemaphoreType.DMA] * (3 + 2 * _NB + 4)
    ),
)
def _pad_kernel(flat_hbm, cu_hbm, len_hbm, zeros_hbm, out_hbm,
                rb0, rb1, rb2, rb3, mbuf, zbuf,
                ix0, ix1, ix2, ix3, midx, zidx, cu_v, len_v,
                sem_cu, sem_len, sem_zb,
                sl0, sl1, sl2, sl3, ss0, ss1, ss2, ss3,
                sem_z, sem_mg, sem_ms, sem_sv):
    bufs = (rb0, rb1, rb2, rb3)
    idxs = (ix0, ix1, ix2, ix3)
    sls = (sl0, sl1, sl2, sl3)
    sss = (ss0, ss1, ss2, ss3)

    c = lax.axis_index("c")
    s = lax.axis_index("s")
    wid = s * 2 + c
    b = wid // 2
    p0 = (wid % 2) * _R

    pltpu.async_copy(cu_hbm, cu_v, sem_cu)
    pltpu.async_copy(len_hbm, len_v, sem_len)
    pltpu.async_copy(zeros_hbm, zbuf, sem_zb)
    pltpu.make_async_copy(cu_hbm, cu_v, sem_cu).wait()
    pltpu.make_async_copy(len_hbm, len_v, sem_len).wait()

    lanes = lax.iota(jnp.int32, 16)
    sel = lanes == b
    cu_b = jnp.sum(jnp.where(sel, cu_v[...], 0))
    len_b = jnp.sum(jnp.where(sel, len_v[...], 0))

    nd = jnp.clip(len_b - p0, 0, _R)
    npad = _R - nd
    base = _m8(wid * _R)
    src0 = cu_b + p0
    nfull = nd // _CH
    rem = nd % _CH
    rm8 = ((rem + 7) // 8) * 8
    sliv = rm8 - rem
    a8 = _m8(base + nfull * _CH + rm8)
    n64 = (npad - sliv) // _CH
    zrem = (npad - sliv) % _CH

    def fill_idx(ref, start):
        for k in range(_CH // 16):
            ref[pl.ds(k * 16, 16)] = jnp.minimum(
                start + k * 16 + lanes, _TOTAL - 1)

    def ld(j):
        return pltpu.make_async_copy(
            flat_hbm.at[idxs[j % _NB]], bufs[j % _NB], sls[j % _NB])

    def st(j):
        return pltpu.make_async_copy(
            bufs[j % _NB],
            out_hbm.at[pl.ds(_m8(base + j * _CH), _CH)], sss[j % _NB])

    for j in range(_NB):
        @pl.when(j < nfull)
        def _(j=j):
            fill_idx(idxs[j], src0 + j * _CH)
            ld(j).start()

    @pl.when(rem > 0)
    def _():
        fill_idx(midx, src0 + nfull * _CH)
        pltpu.async_copy(flat_hbm.at[midx], mbuf, sem_mg)

    pltpu.make_async_copy(zeros_hbm, zbuf, sem_zb).wait()
    zdescs = []
    for j in range(_NCH):
        d = pltpu.make_async_copy(
            zbuf, out_hbm.at[pl.ds(_m8(a8 + j * _CH), _CH)], sem_z)
        zdescs.append((j < n64, d))
    zoff = a8 + n64 * _CH
    for sz in _ZTAIL:
        bit = (zrem // sz) % 2
        d = pltpu.make_async_copy(
            zbuf.at[pl.ds(0, sz)], out_hbm.at[pl.ds(_m8(zoff), sz)], sem_z)
        zdescs.append((bit == 1, d))
        zoff = zoff + bit * sz
    for cond, d in zdescs:
        @pl.when(cond)
        def _(d=d):
            d.start()

    for j in range(_NCH):
        @pl.when(j < nfull)
        def _(j=j):
            ld(j).wait()
            st(j).start()
        if j + _NB < _NCH:
            @pl.when(j + _NB < nfull)
            def _(j=j):
                st(j).wait()
                fill_idx(idxs[(j + _NB) % _NB], src0 + (j + _NB) * _CH)
                ld(j + _NB).start()

    mdescs = []
    moff = 0
    for sz in _MTAIL:
        bit = (rm8 // sz) % 2
        d = pltpu.make_async_copy(
            mbuf.at[pl.ds(moff, sz)],
            out_hbm.at[pl.ds(_m8(base + nfull * _CH + moff), sz)], sem_ms)
        mdescs.append((bit == 1, d))
        moff = moff + bit * sz
    @pl.when(rem > 0)
    def _():
        pltpu.make_async_copy(flat_hbm.at[midx], mbuf, sem_mg).wait()
    for cond, d in mdescs:
        @pl.when(cond)
        def _(d=d):
            d.start()
    for cond, d in mdescs:
        @pl.when(cond)
        def _(d=d):
            d.wait()
    @pl.when(sliv > 0)
    def _():
        zidx[...] = base + nd + jnp.minimum(lanes, sliv - 1)
        pltpu.async_copy(zbuf.at[pl.ds(0, 16)], out_hbm.at[zidx], sem_sv)
        pltpu.make_async_copy(
            zbuf.at[pl.ds(0, 16)], out_hbm.at[zidx], sem_sv).wait()

    for j in range(_NCH):
        @pl.when(jnp.logical_and(j < nfull, j + _NB >= nfull))
        def _(j=j):
            st(j).wait()
    for cond, d in zdescs:
        @pl.when(cond)
        def _(d=d):
            d.wait()


def kernel(flat, cu_seqlens):
    cu16 = cu_seqlens[:_B].astype(jnp.int32)
    lengths_i32 = (cu_seqlens[1:] - cu_seqlens[:-1]).astype(jnp.int32)
    zeros = jnp.zeros((_CH, _D), jnp.float32)
    out = _pad_kernel(flat, cu16, lengths_i32, zeros)
    padded = out.reshape(_B, _MAX_LEN, _D)
    lengths = lengths_i32.astype(jnp.int64)
    return padded, lengths


# speedup vs baseline: 4.5586x; 1.0313x over previous
import functools

import jax
import jax.numpy as jnp
from jax import lax
from jax.experimental import pallas as pl
from jax.experimental.pallas import tpu as pltpu
from jax.experimental.pallas import tpu_sc as plsc

_B = 16
_MAX_LEN = 2048
_TOTAL = 16384
_D = 256
_NW = 32
_R = _B * _MAX_LEN // _NW
_CH = 64
_NCH = _R // _CH
_NB = 4
_ZTAIL = (32, 16, 8)
_MTAIL = (64, 32, 16, 8)

_mesh = plsc.VectorSubcoreMesh(core_axis_name="c", subcore_axis_name="s")


def _m8(x):
    return pl.multiple_of(x, 8)


@functools.partial(
    pl.kernel,
    mesh=_mesh,
    compiler_params=pltpu.CompilerParams(needs_layout_passes=False),
    out_type=jax.ShapeDtypeStruct((_B * _MAX_LEN, _D), jnp.float32),
    scratch_types=(
        [pltpu.VMEM((_CH, _D), jnp.float32)] * _NB
        + [pltpu.VMEM((_CH, _D), jnp.float32)]
        + [pltpu.VMEM((_CH, _D), jnp.float32)]
        + [pltpu.VMEM((_CH,), jnp.int32)] * _NCH
        + [pltpu.VMEM((_CH,), jnp.int32)]
        + [pltpu.VMEM((16,), jnp.int32)] * 2
        + [pltpu.SemaphoreType.DMA] * (3 + 2 * _NB + 3)
    ),
)
def _pad_kernel(flat_hbm, cu_hbm, len_hbm, zeros_hbm, out_hbm,
                rb0, rb1, rb2, rb3, mbuf, zbuf,
                ix0, ix1, ix2, ix3, ix4, ix5, ix6, ix7,
                ix8, ix9, ix10, ix11, ix12, ix13, ix14, ix15,
                midx, cu_v, len_v,
                sem_cu, sem_len, sem_zb,
                sl0, sl1, sl2, sl3, ss0, ss1, ss2, ss3,
                sem_z, sem_mg, sem_ms):
    bufs = (rb0, rb1, rb2, rb3)
    idxs = (ix0, ix1, ix2, ix3, ix4, ix5, ix6, ix7,
            ix8, ix9, ix10, ix11, ix12, ix13, ix14, ix15)
    sls = (sl0, sl1, sl2, sl3)
    sss = (ss0, ss1, ss2, ss3)

    c = lax.axis_index("c")
    s = lax.axis_index("s")
    b = s
    p0 = ((s + c) % 2) * _R

    pltpu.async_copy(cu_hbm, cu_v, sem_cu)
    pltpu.async_copy(len_hbm, len_v, sem_len)
    pltpu.async_copy(zeros_hbm, zbuf, sem_zb)
    pltpu.make_async_copy(cu_hbm, cu_v, sem_cu).wait()
    pltpu.make_async_copy(len_hbm, len_v, sem_len).wait()

    lanes = lax.iota(jnp.int32, 16)
    sel = lanes == b
    cu_b = jnp.sum(jnp.where(sel, cu_v[...], 0))
    len_b = jnp.sum(jnp.where(sel, len_v[...], 0))

    nd = jnp.clip(len_b - p0, 0, _R)
    npad = _R - nd
    base = _m8(b * _MAX_LEN + p0)
    src0 = cu_b + p0
    nfull = nd // _CH
    rem = nd % _CH
    rm8 = ((rem + 7) // 8) * 8
    a8 = _m8(base + nfull * _CH + rm8)
    n64 = (npad - (rm8 - rem)) // _CH
    zrem = (npad - (rm8 - rem)) % _CH

    def fill_idx(ref, start):
        for k in range(_CH // 16):
            ref[pl.ds(k * 16, 16)] = jnp.minimum(
                start + k * 16 + lanes, _TOTAL - 1)

    def ld(j):
        return pltpu.make_async_copy(
            flat_hbm.at[idxs[j]], bufs[j % _NB], sls[j % _NB])

    def st(j):
        return pltpu.make_async_copy(
            bufs[j % _NB],
            out_hbm.at[pl.ds(_m8(base + j * _CH), _CH)], sss[j % _NB])

    for j in range(_NCH):
        fill_idx(idxs[j], src0 + j * _CH)
    fill_idx(midx, src0 + nfull * _CH)
    idx_gate = jnp.sum(midx[pl.ds(3 * 16, 16)]) > -1

    for j in range(_NB):
        @pl.when(jnp.logical_and(idx_gate, j < nfull))
        def _(j=j):
            ld(j).start()

    @pl.when(jnp.logical_and(idx_gate, rem > 0))
    def _():
        pltpu.async_copy(flat_hbm.at[midx], mbuf, sem_mg)

    pltpu.make_async_copy(zeros_hbm, zbuf, sem_zb).wait()
    zdescs = []
    for j in range(_NCH):
        d = pltpu.make_async_copy(
            zbuf, out_hbm.at[pl.ds(_m8(a8 + j * _CH), _CH)], sem_z)
        zdescs.append((j < n64, d))
    zoff = a8 + n64 * _CH
    for sz in _ZTAIL:
        bit = (zrem // sz) % 2
        d = pltpu.make_async_copy(
            zbuf.at[pl.ds(0, sz)], out_hbm.at[pl.ds(_m8(zoff), sz)], sem_z)
        zdescs.append((bit == 1, d))
        zoff = zoff + bit * sz
    for cond, d in zdescs:
        @pl.when(cond)
        def _(d=d):
            d.start()

    for j in range(_NCH):
        @pl.when(j < nfull)
        def _(j=j):
            ld(j).wait()
            st(j).start()
        if j + _NB < _NCH:
            @pl.when(j + _NB < nfull)
            def _(j=j):
                st(j).wait()
                ld(j + _NB).start()

    zero16 = jnp.zeros((16,), jnp.float32)

    @pl.when(rem > 0)
    def _():
        pltpu.make_async_copy(flat_hbm.at[midx], mbuf, sem_mg).wait()

        def zrow(r, carry):
            for k in range(_D // 16):
                mbuf[r, pl.ds(k * 16, 16)] = zero16
            return carry
        lax.fori_loop(rem, rm8, zrow, 0)
    sliv = rm8 - rem
    grow = jnp.maximum(rm8 - 1, 0)
    mix_gate = jnp.sum(mbuf[grow, pl.ds(_D - 16, 16)]) < 1.0
    mdescs = []
    moff = 0
    for sz in _MTAIL:
        bit = (rm8 // sz) % 2
        d = pltpu.make_async_copy(
            mbuf.at[pl.ds(moff, sz)],
            out_hbm.at[pl.ds(_m8(base + nfull * _CH + moff), sz)], sem_ms)
        mdescs.append((bit == 1, d))
        moff = moff + bit * sz
    for cond, d in mdescs:
        @pl.when(jnp.logical_and(
            cond, jnp.logical_and(sliv > 0, mix_gate)))
        def _(d=d):
            d.start()
        @pl.when(jnp.logical_and(cond, sliv == 0))
        def _(d=d):
            d.start()
    for cond, d in mdescs:
        @pl.when(cond)
        def _(d=d):
            d.wait()

    for j in range(_NCH):
        @pl.when(jnp.logical_and(j < nfull, j + _NB >= nfull))
        def _(j=j):
            st(j).wait()
    for cond, d in zdescs:
        @pl.when(cond)
        def _(d=d):
            d.wait()


def kernel(flat, cu_seqlens):
    cu16 = cu_seqlens[:_B].astype(jnp.int32)
    lengths_i32 = (cu_seqlens[1:] - cu_seqlens[:-1]).astype(jnp.int32)
    zeros = jnp.zeros((_CH, _D), jnp.float32)
    out = _pad_kernel(flat, cu16, lengths_i32, zeros)
    padded = out.reshape(_B, _MAX_LEN, _D)
    lengths = lengths_i32.astype(jnp.int64)
    return padded, lengths


# speedup vs baseline: 4.7250x; 1.0365x over previous
import functools

import jax
import jax.numpy as jnp
from jax import lax
from jax.experimental import pallas as pl
from jax.experimental.pallas import tpu as pltpu
from jax.experimental.pallas import tpu_sc as plsc

_B = 16
_MAX_LEN = 2048
_TOTAL = 16384
_D = 256
_NW = 32
_R = _B * _MAX_LEN // _NW
_CH = 64
_NCH = _R // _CH
_NB = 4
_ZTAIL = (32, 16, 8)
_MTAIL = (64, 32, 16, 8)

_mesh = plsc.VectorSubcoreMesh(core_axis_name="c", subcore_axis_name="s")


def _m8(x):
    return pl.multiple_of(x, 8)


@functools.partial(
    pl.kernel,
    mesh=_mesh,
    compiler_params=pltpu.CompilerParams(needs_layout_passes=False),
    out_type=jax.ShapeDtypeStruct((_B * _MAX_LEN, _D), jnp.float32),
    scratch_types=(
        [pltpu.VMEM((_CH, _D), jnp.float32)] * _NB
        + [pltpu.VMEM((_CH, _D), jnp.float32)]
        + [pltpu.VMEM((_CH, _D), jnp.float32)]
        + [pltpu.VMEM((_NCH, _CH), jnp.int32)]
        + [pltpu.VMEM((_CH,), jnp.int32)]
        + [pltpu.VMEM((16,), jnp.int32)] * 2
        + [pltpu.SemaphoreType.DMA] * (3 + 2 * _NB + 3)
    ),
)
def _pad_kernel(flat_hbm, cu_hbm, len_hbm, zeros_hbm, out_hbm,
                rb0, rb1, rb2, rb3, mbuf, zbuf,
                gidx, midx, cu_v, len_v,
                sem_cu, sem_len, sem_zb,
                sl0, sl1, sl2, sl3, ss0, ss1, ss2, ss3,
                sem_z, sem_mg, sem_ms):
    bufs = (rb0, rb1, rb2, rb3)
    sls = (sl0, sl1, sl2, sl3)
    sss = (ss0, ss1, ss2, ss3)

    c = lax.axis_index("c")
    s = lax.axis_index("s")
    b = s
    p0 = ((s + c) % 2) * _R

    pltpu.async_copy(cu_hbm, cu_v, sem_cu)
    pltpu.async_copy(len_hbm, len_v, sem_len)
    pltpu.async_copy(zeros_hbm, zbuf, sem_zb)
    pltpu.make_async_copy(cu_hbm, cu_v, sem_cu).wait()
    pltpu.make_async_copy(len_hbm, len_v, sem_len).wait()

    lanes = lax.iota(jnp.int32, 16)
    sel = lanes == b
    cu_b = jnp.sum(jnp.where(sel, cu_v[...], 0))
    len_b = jnp.sum(jnp.where(sel, len_v[...], 0))

    nd = jnp.clip(len_b - p0, 0, _R)
    npad = _R - nd
    base = _m8(b * _MAX_LEN + p0)
    src0 = cu_b + p0
    nfull = nd // _CH
    rem = nd % _CH
    rm8 = ((rem + 7) // 8) * 8
    sliv = rm8 - rem
    a8 = _m8(base + nfull * _CH + rm8)
    n64 = (npad - sliv) // _CH
    zrem = (npad - sliv) % _CH
    ngrp = nfull // _NB

    def fill_row(j, carry):
        for k in range(_CH // 16):
            gidx[j, pl.ds(k * 16, 16)] = jnp.minimum(
                src0 + j * _CH + k * 16 + lanes, _TOTAL - 1)
        return carry
    lax.fori_loop(0, _NCH, fill_row, 0)
    for k in range(_CH // 16):
        midx[pl.ds(k * 16, 16)] = jnp.minimum(
            src0 + nfull * _CH + k * 16 + lanes, _TOTAL - 1)
    idx_gate = jnp.sum(midx[pl.ds(3 * 16, 16)]) > -1

    def ld(t, j):
        return pltpu.make_async_copy(
            flat_hbm.at[gidx.at[j]], bufs[t], sls[t])

    def st(t, j):
        return pltpu.make_async_copy(
            bufs[t],
            out_hbm.at[pl.ds(_m8(base + j * _CH), _CH)], sss[t])

    for t in range(_NB):
        @pl.when(jnp.logical_and(idx_gate, t < nfull))
        def _(t=t):
            ld(t, t).start()

    @pl.when(jnp.logical_and(idx_gate, rem > 0))
    def _():
        pltpu.async_copy(flat_hbm.at[midx], mbuf, sem_mg)

    pltpu.make_async_copy(zeros_hbm, zbuf, sem_zb).wait()

    def zfire(i, carry):
        pltpu.async_copy(
            zbuf, out_hbm.at[pl.ds(_m8(a8 + i * _CH), _CH)], sem_z)
        return carry
    lax.fori_loop(0, n64, zfire, 0)
    zdescs = []
    zoff = a8 + n64 * _CH
    for sz in _ZTAIL:
        bit = (zrem // sz) % 2
        d = pltpu.make_async_copy(
            zbuf.at[pl.ds(0, sz)], out_hbm.at[pl.ds(_m8(zoff), sz)], sem_z)
        zdescs.append((bit == 1, d))
        zoff = zoff + bit * sz
    for cond, d in zdescs:
        @pl.when(cond)
        def _(d=d):
            d.start()

    def group(g, carry):
        for t in range(_NB):
            j = g * _NB + t
            ld(t, j).wait()
            st(t, j).start()
            @pl.when(j + _NB < nfull)
            def _(t=t, j=j):
                st(t, j).wait()
                ld(t, j + _NB).start()
        return carry
    lax.fori_loop(0, ngrp, group, 0)

    for t in range(_NB - 1):
        @pl.when(ngrp * _NB + t < nfull)
        def _(t=t):
            j = ngrp * _NB + t
            ld(t, j).wait()
            st(t, j).start()

    zero16 = jnp.zeros((16,), jnp.float32)

    @pl.when(rem > 0)
    def _():
        pltpu.make_async_copy(flat_hbm.at[midx], mbuf, sem_mg).wait()

        def zrow(r, carry):
            for k in range(_D // 16):
                mbuf[r, pl.ds(k * 16, 16)] = zero16
            return carry
        lax.fori_loop(rem, rm8, zrow, 0)
    grow = jnp.maximum(rm8 - 1, 0)
    mix_gate = jnp.sum(mbuf[grow, pl.ds(_D - 16, 16)]) < 1.0
    mdescs = []
    moff = 0
    for sz in _MTAIL:
        bit = (rm8 // sz) % 2
        d = pltpu.make_async_copy(
            mbuf.at[pl.ds(moff, sz)],
            out_hbm.at[pl.ds(_m8(base + nfull * _CH + moff), sz)], sem_ms)
        mdescs.append((bit == 1, d))
        moff = moff + bit * sz
    for cond, d in mdescs:
        @pl.when(jnp.logical_and(
            cond, jnp.logical_and(sliv > 0, mix_gate)))
        def _(d=d):
            d.start()
        @pl.when(jnp.logical_and(cond, sliv == 0))
        def _(d=d):
            d.start()
    for cond, d in mdescs:
        @pl.when(cond)
        def _(d=d):
            d.wait()

    for t in range(_NB):
        @pl.when(t < nfull)
        def _(t=t):
            lj = ((nfull - 1 - t) // _NB) * _NB + t
            st(t, lj).wait()

    def zdrain(i, carry):
        pltpu.make_async_copy(
            zbuf, out_hbm.at[pl.ds(_m8(a8 + i * _CH), _CH)], sem_z).wait()
        return carry
    lax.fori_loop(0, n64, zdrain, 0)
    for cond, d in zdescs:
        @pl.when(cond)
        def _(d=d):
            d.wait()


def kernel(flat, cu_seqlens):
    cu16 = cu_seqlens[:_B].astype(jnp.int32)
    lengths_i32 = (cu_seqlens[1:] - cu_seqlens[:-1]).astype(jnp.int32)
    zeros = jnp.zeros((_CH, _D), jnp.float32)
    out = _pad_kernel(flat, cu16, lengths_i32, zeros)
    padded = out.reshape(_B, _MAX_LEN, _D)
    lengths = lengths_i32.astype(jnp.int64)
    return padded, lengths
